# baseline (speedup 1.0000x reference)
import functools

import jax
import jax.numpy as jnp
from jax import lax
from jax.experimental import pallas as pl
from jax.experimental.pallas import tpu as pltpu
from jax.experimental.pallas import tpu_sc as plsc

N_NODES = 10000
N_EDGES = 320000
CH = 128
B = 128
NCHUNK = N_EDGES // B
NC, NS = 2, 16
NW = NC * NS
RPT = N_NODES // NS


NBUF = 2


def _sc_agg(split_channels: bool, with_counts: bool):
    NWK = NW if not split_channels else NS
    base_nk, extra = NCHUNK // NWK, NCHUNK % NWK
    nk_max = base_nk + (1 if extra else 0)

    mesh = plsc.VectorSubcoreMesh(core_axis_name="c", subcore_axis_name="s")
    out_type = [jax.ShapeDtypeStruct((NC, N_NODES, CH), jnp.float32)]
    if with_counts:
        out_type.append(jax.ShapeDtypeStruct((NC, N_NODES, 16), jnp.float32))
    scratch = [
        pltpu.VMEM_SHARED((N_NODES, CH), jnp.float32),
        pltpu.VMEM_SHARED((N_NODES, 16), jnp.float32),
        pltpu.VMEM((4, B), jnp.int32),
        pltpu.VMEM((4, B), jnp.int32),
        [pltpu.VMEM((B, CH), jnp.float32) for _ in range(2)],
        pltpu.VMEM((B, 16), jnp.float32),
        [pltpu.SemaphoreType.DMA for _ in range(2)],
        [pltpu.SemaphoreType.DMA for _ in range(2)],
        [pltpu.SemaphoreType.DMA for _ in range(2)],
    ]

    def body(x_hbm, src_hbm, dst_hbm, zrow_hbm, zcnt_hbm, ones_hbm,
             *rest):
        if with_counts:
            out_hbm, cnt_hbm = rest[0], rest[1]
            acc, cacc, sidx, didx, rows, obuf, gsem, ssem, isem = rest[2:]
        else:
            out_hbm = rest[0]
            acc, cacc, sidx, didx, rows, obuf, gsem, ssem, isem = rest[1:]
        c = lax.axis_index("c")
        s = lax.axis_index("s")
        w = s * NC + c if not split_channels else s
        start = w * base_nk + jnp.minimum(w, extra)
        nk = base_nk + jnp.where(w < extra, 1, 0) if extra else base_nk
        xsrc = x_hbm.at[c] if split_channels else x_hbm

        pltpu.sync_copy(zrow_hbm, acc.at[pl.ds(s * RPT, RPT)])
        if with_counts:
            pltpu.sync_copy(zcnt_hbm, cacc.at[pl.ds(s * RPT, RPT)])
            pltpu.sync_copy(ones_hbm, obuf)
        pltpu.sync_copy(src_hbm.at[pl.ds(start, 2)], sidx.at[pl.ds(0, 2)])
        pltpu.sync_copy(dst_hbm.at[pl.ds(start, 2)], didx.at[pl.ds(0, 2)])
        pltpu.async_copy(xsrc.at[sidx.at[0]], rows[0], gsem[0])
        plsc.subcore_barrier()

        def scatter_wait(j, p, q):
            pltpu.make_async_copy(rows[q], acc.at[didx.at[p]],
                                  ssem[q]).wait()
            if with_counts:
                pltpu.make_async_copy(obuf, cacc.at[didx.at[p]],
                                      ssem[q]).wait()

        def step(j, t):
            q = t % 2

            @pl.when(j < nk)
            def _():
                if t % 2 == 1:
                    @pl.when(j + 1 < nk)
                    def _():
                        cb = jnp.minimum(start + j + 1, NCHUNK - 2)
                        pltpu.make_async_copy(
                            src_hbm.at[pl.ds(cb, 2)],
                            sidx.at[pl.ds((t + 1) % 4, 2)], isem[0]).wait()
                        pltpu.make_async_copy(
                            dst_hbm.at[pl.ds(cb, 2)],
                            didx.at[pl.ds((t + 1) % 4, 2)], isem[1]).wait()

                pltpu.make_async_copy(xsrc.at[sidx.at[t]], rows[q],
                                      gsem[q]).wait()
                pltpu.async_copy(rows[q], acc.at[didx.at[t]], ssem[q],
                                 add=True)
                if with_counts:
                    pltpu.async_copy(obuf, cacc.at[didx.at[t]], ssem[q],
                                     add=True)

                @pl.when(j >= 1)
                def _():
                    scatter_wait(j - 1, (t - 1) % 4, 1 - q)
                if t % 2 == 0:
                    @pl.when(j + 2 < nk)
                    def _():
                        cb = jnp.minimum(start + j + 2, NCHUNK - 2)
                        pltpu.async_copy(src_hbm.at[pl.ds(cb, 2)],
                                         sidx.at[pl.ds((t + 2) % 4, 2)],
                                         isem[0])
                        pltpu.async_copy(dst_hbm.at[pl.ds(cb, 2)],
                                         didx.at[pl.ds((t + 2) % 4, 2)],
                                         isem[1])

                @pl.when(j + 1 < nk)
                def _():
                    pltpu.async_copy(xsrc.at[sidx.at[(t + 1) % 4]],
                                     rows[1 - q], gsem[1 - q])

        def quad(u, carry):
            for t in range(4):
                step(u * 4 + t, t)
            return carry

        lax.fori_loop(0, (nk_max + 3) // 4, quad, 0)

        for q in range(2):
            @pl.when((nk % 2) == (1 - q))
            def _(q=q):
                scatter_wait(nk - 1, (nk - 1) % 4, q)

        plsc.subcore_barrier()
        pltpu.sync_copy(acc.at[pl.ds(s * RPT, RPT)],
                        out_hbm.at[c].at[pl.ds(s * RPT, RPT)])
        if with_counts:
            pltpu.sync_copy(cacc.at[pl.ds(s * RPT, RPT)],
                            cnt_hbm.at[c].at[pl.ds(s * RPT, RPT)])

    return pl.kernel(body, out_type=out_type, mesh=mesh,
                     scratch_types=scratch,
                     compiler_params=pltpu.CompilerParams(
                         use_tc_tiling_on_sc=False))


_sc_agg_edges_counts = _sc_agg(split_channels=False, with_counts=True)
_sc_agg_edges = _sc_agg(split_channels=False, with_counts=False)
_sc_agg_split = _sc_agg(split_channels=True, with_counts=False)


def _inv_deg(cnt_ref):
    deg = cnt_ref[0] + cnt_ref[1]
    return 1.0 / jnp.clip(deg[:, 0:1], 1.0, None)


BM = 1000
NB = N_NODES // BM


def _layer0_body(agg_ref, cnt_ref, x_ref, wl_ref, bl_ref, wr_ref, out_ref):
    inv = _inv_deg(cnt_ref)
    aggm = (agg_ref[0] + agg_ref[1]) * inv
    res = jnp.dot(aggm, wl_ref[...], preferred_element_type=jnp.float32)
    res = res + jnp.dot(x_ref[...], wr_ref[...],
                        preferred_element_type=jnp.float32)
    res = jnp.maximum(res + bl_ref[...], 0.0)
    out_ref[0] = res[:, :CH]
    out_ref[1] = res[:, CH:]


def _layer0_tc(agg, cnt, x, WlT, bl, WrT):
    return pl.pallas_call(
        _layer0_body,
        grid=(NB,),
        in_specs=[
            pl.BlockSpec((NC, BM, CH), lambda i: (0, i, 0)),
            pl.BlockSpec((NC, BM, 16), lambda i: (0, i, 0)),
            pl.BlockSpec((BM, CH), lambda i: (i, 0)),
            pl.BlockSpec((CH, 2 * CH), lambda i: (0, 0)),
            pl.BlockSpec((1, 2 * CH), lambda i: (0, 0)),
            pl.BlockSpec((CH, 2 * CH), lambda i: (0, 0)),
        ],
        out_specs=pl.BlockSpec((NC, BM, CH), lambda i: (0, i, 0)),
        out_shape=jax.ShapeDtypeStruct((NC, N_NODES, CH), jnp.float32),
    )(agg, cnt, x, WlT, bl, WrT)


def _layer1_body(agg_ref, cnt_ref, x1_ref, wl1_ref, bl1_ref, wr1_ref,
                 wl2_ref, wr2_ref, bl2_ref, h_ref, r2_ref):
    inv = _inv_deg(cnt_ref)
    x2 = jnp.dot(agg_ref[0] * inv, wl1_ref[:CH, :],
                 preferred_element_type=jnp.float32)
    x2 = x2 + jnp.dot(agg_ref[1] * inv, wl1_ref[CH:, :],
                      preferred_element_type=jnp.float32)
    x2 = x2 + jnp.dot(x1_ref[0], wr1_ref[:CH, :],
                      preferred_element_type=jnp.float32)
    x2 = x2 + jnp.dot(x1_ref[1], wr1_ref[CH:, :],
                      preferred_element_type=jnp.float32)
    x2 = jnp.maximum(x2 + bl1_ref[...], 0.0)
    h_ref[...] = jnp.dot(x2, wl2_ref[...], preferred_element_type=jnp.float32)
    r2_ref[...] = jnp.dot(x2, wr2_ref[...],
                          preferred_element_type=jnp.float32) + bl2_ref[...]


def _layer1_tc(agg, cnt, x1s, Wl1T, bl1, Wr1T, Wl2T, Wr2T, bl2):
    return pl.pallas_call(
        _layer1_body,
        grid=(NB,),
        in_specs=[
            pl.BlockSpec((NC, BM, CH), lambda i: (0, i, 0)),
            pl.BlockSpec((NC, BM, 16), lambda i: (0, i, 0)),
            pl.BlockSpec((NC, BM, CH), lambda i: (0, i, 0)),
            pl.BlockSpec((2 * CH, 2 * CH), lambda i: (0, 0)),
            pl.BlockSpec((1, 2 * CH), lambda i: (0, 0)),
            pl.BlockSpec((2 * CH, 2 * CH), lambda i: (0, 0)),
            pl.BlockSpec((2 * CH, CH), lambda i: (0, 0)),
            pl.BlockSpec((2 * CH, CH), lambda i: (0, 0)),
            pl.BlockSpec((1, CH), lambda i: (0, 0)),
        ],
        out_specs=[
            pl.BlockSpec((BM, CH), lambda i: (i, 0)),
            pl.BlockSpec((BM, CH), lambda i: (i, 0)),
        ],
        out_shape=[
            jax.ShapeDtypeStruct((N_NODES, CH), jnp.float32),
            jax.ShapeDtypeStruct((N_NODES, CH), jnp.float32),
        ],
    )(agg, cnt, x1s, Wl1T, bl1, Wr1T, Wl2T, Wr2T, bl2)


def _final_body(p_ref, cnt_ref, r2_ref, out_ref):
    inv = _inv_deg(cnt_ref)
    out_ref[...] = jnp.maximum((p_ref[0] + p_ref[1]) * inv + r2_ref[...], 0.0)


def _final_tc(p, cnt, r2):
    return pl.pallas_call(
        _final_body,
        grid=(NB,),
        in_specs=[
            pl.BlockSpec((NC, BM, CH), lambda i: (0, i, 0)),
            pl.BlockSpec((NC, BM, 16), lambda i: (0, i, 0)),
            pl.BlockSpec((BM, CH), lambda i: (i, 0)),
        ],
        out_specs=pl.BlockSpec((BM, CH), lambda i: (i, 0)),
        out_shape=jax.ShapeDtypeStruct((N_NODES, CH), jnp.float32),
    )(p, cnt, r2)


def kernel(x, edge, num_sampled_nodes, num_sampled_edges,
           Wl0, bl0, Wr0, Wl1, bl1, Wr1, Wl2, bl2, Wr2):
    src = edge[0].reshape(NCHUNK, B)
    dst = edge[1].reshape(NCHUNK, B)
    zrow = jnp.zeros((RPT, CH), jnp.float32)
    zcnt = jnp.zeros((RPT, 16), jnp.float32)
    ones = jnp.ones((B, 16), jnp.float32)

    p0, cnt = _sc_agg_edges_counts(x, src, dst, zrow, zcnt, ones)
    x1s = _layer0_tc(p0, cnt, x, Wl0.T, bl0.reshape(1, -1), Wr0.T)

    (a1,) = _sc_agg_split(x1s, src, dst, zrow, zcnt, ones)
    h, r2 = _layer1_tc(a1, cnt, x1s, Wl1.T, bl1.reshape(1, -1), Wr1.T,
                       Wl2.T, Wr2.T, bl2.reshape(1, -1))

    (p2,) = _sc_agg_edges(h, src, dst, zrow, zcnt, ones)
    return _final_tc(p2, cnt, r2)


# speedup vs baseline: 1.0015x; 1.0015x over previous
import functools

import jax
import jax.numpy as jnp
from jax import lax
from jax.experimental import pallas as pl
from jax.experimental.pallas import tpu as pltpu
from jax.experimental.pallas import tpu_sc as plsc

N_NODES = 10000
N_EDGES = 320000
CH = 128
B = 128
NCHUNK = N_EDGES // B
NC, NS = 2, 16
NW = NC * NS
RPT = N_NODES // NS


NBUF = 2


def _sc_agg(split_channels: bool, with_counts: bool):
    NWK = NW if not split_channels else NS
    base_nk, extra = NCHUNK // NWK, NCHUNK % NWK
    nk_max = base_nk + (1 if extra else 0)

    mesh = plsc.VectorSubcoreMesh(core_axis_name="c", subcore_axis_name="s")
    out_type = [jax.ShapeDtypeStruct((NC, N_NODES, CH), jnp.float32)]
    if with_counts:
        out_type.append(jax.ShapeDtypeStruct((NC, N_NODES, 16), jnp.float32))
    scratch = [
        pltpu.VMEM_SHARED((N_NODES, CH), jnp.float32),
        pltpu.VMEM_SHARED((N_NODES, 16), jnp.float32),
        pltpu.VMEM((4, B), jnp.int32),
        pltpu.VMEM((4, B), jnp.int32),
        [pltpu.VMEM((B, CH), jnp.float32) for _ in range(2)],
        pltpu.VMEM((B, 16), jnp.float32),
        [pltpu.SemaphoreType.DMA for _ in range(2)],
        [pltpu.SemaphoreType.DMA for _ in range(2)],
        [pltpu.SemaphoreType.DMA for _ in range(2)],
    ]

    def body(x_hbm, src_hbm, dst_hbm, zrow_hbm, zcnt_hbm, ones_hbm,
             *rest):
        if with_counts:
            out_hbm, cnt_hbm = rest[0], rest[1]
            acc, cacc, sidx, didx, rows, obuf, gsem, ssem, isem = rest[2:]
        else:
            out_hbm = rest[0]
            acc, cacc, sidx, didx, rows, obuf, gsem, ssem, isem = rest[1:]
        c = lax.axis_index("c")
        s = lax.axis_index("s")
        w = s * NC + c if not split_channels else s
        start = w * base_nk + jnp.minimum(w, extra)
        nk = base_nk + jnp.where(w < extra, 1, 0) if extra else base_nk
        xsrc = x_hbm.at[c] if split_channels else x_hbm

        pltpu.sync_copy(zrow_hbm, acc.at[pl.ds(s * RPT, RPT)])
        if with_counts:
            pltpu.sync_copy(zcnt_hbm, cacc.at[pl.ds(s * RPT, RPT)])
            pltpu.sync_copy(ones_hbm, obuf)
        pltpu.sync_copy(src_hbm.at[pl.ds(start, 2)], sidx.at[pl.ds(0, 2)])
        pltpu.sync_copy(dst_hbm.at[pl.ds(start, 2)], didx.at[pl.ds(0, 2)])
        pltpu.async_copy(xsrc.at[sidx.at[0]], rows[0], gsem[0])
        plsc.subcore_barrier()

        def scatter_wait(j, p, q):
            pltpu.make_async_copy(rows[q], acc.at[didx.at[p]],
                                  ssem[q]).wait()
            if with_counts:
                pltpu.make_async_copy(obuf, cacc.at[didx.at[p]],
                                      ssem[q]).wait()

        def step(j, t):
            q = t % 2

            @pl.when(j < nk)
            def _():
                if t % 2 == 1:
                    @pl.when(j + 1 < nk)
                    def _():
                        cb = jnp.minimum(start + j + 1, NCHUNK - 2)
                        pltpu.make_async_copy(
                            src_hbm.at[pl.ds(cb, 2)],
                            sidx.at[pl.ds((t + 1) % 4, 2)], isem[0]).wait()
                        pltpu.make_async_copy(
                            dst_hbm.at[pl.ds(cb, 2)],
                            didx.at[pl.ds((t + 1) % 4, 2)], isem[1]).wait()

                pltpu.make_async_copy(xsrc.at[sidx.at[t]], rows[q],
                                      gsem[q]).wait()
                pltpu.async_copy(rows[q], acc.at[didx.at[t]], ssem[q],
                                 add=True)
                if with_counts:
                    pltpu.async_copy(obuf, cacc.at[didx.at[t]], ssem[q],
                                     add=True)

                @pl.when(j >= 1)
                def _():
                    scatter_wait(j - 1, (t - 1) % 4, 1 - q)
                if t % 2 == 0:
                    @pl.when(j + 2 < nk)
                    def _():
                        cb = jnp.minimum(start + j + 2, NCHUNK - 2)
                        pltpu.async_copy(src_hbm.at[pl.ds(cb, 2)],
                                         sidx.at[pl.ds((t + 2) % 4, 2)],
                                         isem[0])
                        pltpu.async_copy(dst_hbm.at[pl.ds(cb, 2)],
                                         didx.at[pl.ds((t + 2) % 4, 2)],
                                         isem[1])

                @pl.when(j + 1 < nk)
                def _():
                    pltpu.async_copy(xsrc.at[sidx.at[(t + 1) % 4]],
                                     rows[1 - q], gsem[1 - q])

        def quad(u, carry):
            for t in range(4):
                step(u * 4 + t, t)
            return carry

        lax.fori_loop(0, (nk_max + 3) // 4, quad, 0)

        for q in range(2):
            @pl.when((nk % 2) == (1 - q))
            def _(q=q):
                scatter_wait(nk - 1, (nk - 1) % 4, q)

        plsc.subcore_barrier()
        pltpu.sync_copy(acc.at[pl.ds(s * RPT, RPT)],
                        out_hbm.at[c].at[pl.ds(s * RPT, RPT)])
        if with_counts:
            pltpu.sync_copy(cacc.at[pl.ds(s * RPT, RPT)],
                            cnt_hbm.at[c].at[pl.ds(s * RPT, RPT)])

    return pl.kernel(body, out_type=out_type, mesh=mesh,
                     scratch_types=scratch,
                     compiler_params=pltpu.CompilerParams(
                         use_tc_tiling_on_sc=False))


_sc_agg_edges_counts = _sc_agg(split_channels=False, with_counts=True)
_sc_agg_edges = _sc_agg(split_channels=False, with_counts=False)
_sc_agg_split = _sc_agg(split_channels=True, with_counts=False)


def _inv_deg(cnt_ref):
    deg = cnt_ref[0] + cnt_ref[1]
    return 1.0 / jnp.clip(deg[:, 0:1], 1.0, None)


BM = 1000
NB = N_NODES // BM


def _dot_t(a, w):
    return lax.dot_general(a, w, (((1,), (1,)), ((), ())),
                           preferred_element_type=jnp.float32)


def _layer0_body(agg_ref, cnt_ref, x_ref, wl_ref, bl_ref, wr_ref, out_ref):
    inv = _inv_deg(cnt_ref)
    aggm = (agg_ref[0] + agg_ref[1]) * inv
    res = _dot_t(aggm, wl_ref[...]) + _dot_t(x_ref[...], wr_ref[...])
    res = jnp.maximum(res + bl_ref[...], 0.0)
    out_ref[0] = res[:, :CH]
    out_ref[1] = res[:, CH:]


def _layer0_tc(agg, cnt, x, WlT, bl, WrT):
    return pl.pallas_call(
        _layer0_body,
        grid=(NB,),
        in_specs=[
            pl.BlockSpec((NC, BM, CH), lambda i: (0, i, 0)),
            pl.BlockSpec((NC, BM, 16), lambda i: (0, i, 0)),
            pl.BlockSpec((BM, CH), lambda i: (i, 0)),
            pl.BlockSpec((2 * CH, CH), lambda i: (0, 0)),
            pl.BlockSpec((1, 2 * CH), lambda i: (0, 0)),
            pl.BlockSpec((2 * CH, CH), lambda i: (0, 0)),
        ],
        out_specs=pl.BlockSpec((NC, BM, CH), lambda i: (0, i, 0)),
        out_shape=jax.ShapeDtypeStruct((NC, N_NODES, CH), jnp.float32),
    )(agg, cnt, x, WlT, bl, WrT)


def _layer1_body(agg_ref, cnt_ref, x1_ref, wl1_ref, bl1_ref, wr1_ref,
                 wl2_ref, wr2_ref, bl2_ref, h_ref, r2_ref):
    inv = _inv_deg(cnt_ref)
    x2 = _dot_t(agg_ref[0] * inv, wl1_ref[:, :CH])
    x2 = x2 + _dot_t(agg_ref[1] * inv, wl1_ref[:, CH:])
    x2 = x2 + _dot_t(x1_ref[0], wr1_ref[:, :CH])
    x2 = x2 + _dot_t(x1_ref[1], wr1_ref[:, CH:])
    x2 = jnp.maximum(x2 + bl1_ref[...], 0.0)
    h_ref[...] = _dot_t(x2, wl2_ref[...])
    r2_ref[...] = _dot_t(x2, wr2_ref[...]) + bl2_ref[...]


def _layer1_tc(agg, cnt, x1s, Wl1T, bl1, Wr1T, Wl2T, Wr2T, bl2):
    return pl.pallas_call(
        _layer1_body,
        grid=(NB,),
        in_specs=[
            pl.BlockSpec((NC, BM, CH), lambda i: (0, i, 0)),
            pl.BlockSpec((NC, BM, 16), lambda i: (0, i, 0)),
            pl.BlockSpec((NC, BM, CH), lambda i: (0, i, 0)),
            pl.BlockSpec((2 * CH, 2 * CH), lambda i: (0, 0)),
            pl.BlockSpec((1, 2 * CH), lambda i: (0, 0)),
            pl.BlockSpec((2 * CH, 2 * CH), lambda i: (0, 0)),
            pl.BlockSpec((CH, 2 * CH), lambda i: (0, 0)),
            pl.BlockSpec((CH, 2 * CH), lambda i: (0, 0)),
            pl.BlockSpec((1, CH), lambda i: (0, 0)),
        ],
        out_specs=[
            pl.BlockSpec((BM, CH), lambda i: (i, 0)),
            pl.BlockSpec((BM, CH), lambda i: (i, 0)),
        ],
        out_shape=[
            jax.ShapeDtypeStruct((N_NODES, CH), jnp.float32),
            jax.ShapeDtypeStruct((N_NODES, CH), jnp.float32),
        ],
    )(agg, cnt, x1s, Wl1T, bl1, Wr1T, Wl2T, Wr2T, bl2)


def _final_body(p_ref, cnt_ref, r2_ref, out_ref):
    inv = _inv_deg(cnt_ref)
    out_ref[...] = jnp.maximum((p_ref[0] + p_ref[1]) * inv + r2_ref[...], 0.0)


def _final_tc(p, cnt, r2):
    return pl.pallas_call(
        _final_body,
        grid=(NB,),
        in_specs=[
            pl.BlockSpec((NC, BM, CH), lambda i: (0, i, 0)),
            pl.BlockSpec((NC, BM, 16), lambda i: (0, i, 0)),
            pl.BlockSpec((BM, CH), lambda i: (i, 0)),
        ],
        out_specs=pl.BlockSpec((BM, CH), lambda i: (i, 0)),
        out_shape=jax.ShapeDtypeStruct((N_NODES, CH), jnp.float32),
    )(p, cnt, r2)


def kernel(x, edge, num_sampled_nodes, num_sampled_edges,
           Wl0, bl0, Wr0, Wl1, bl1, Wr1, Wl2, bl2, Wr2):
    src = edge[0].reshape(NCHUNK, B)
    dst = edge[1].reshape(NCHUNK, B)
    zrow = jnp.zeros((RPT, CH), jnp.float32)
    zcnt = jnp.zeros((RPT, 16), jnp.float32)
    ones = jnp.ones((B, 16), jnp.float32)

    p0, cnt = _sc_agg_edges_counts(x, src, dst, zrow, zcnt, ones)
    x1s = _layer0_tc(p0, cnt, x, Wl0, bl0.reshape(1, -1), Wr0)

    (a1,) = _sc_agg_split(x1s, src, dst, zrow, zcnt, ones)
    h, r2 = _layer1_tc(a1, cnt, x1s, Wl1, bl1.reshape(1, -1), Wr1,
                       Wl2, Wr2, bl2.reshape(1, -1))

    (p2,) = _sc_agg_edges(h, src, dst, zrow, zcnt, ones)
    return _final_tc(p2, cnt, r2)


# speedup vs baseline: 1.1994x; 1.1976x over previous
import functools

import jax
import jax.numpy as jnp
from jax import lax
from jax.experimental import pallas as pl
from jax.experimental.pallas import tpu as pltpu
from jax.experimental.pallas import tpu_sc as plsc

N_NODES = 10000
N_EDGES = 320000
CH = 128
B = 128
NCHUNK = N_EDGES // B
NC, NS = 2, 16
NW = NC * NS
RPT = N_NODES // NS


NBUF = 2


def _sc_agg(split_channels: bool, with_counts: bool):
    NWK = NW if not split_channels else NS
    base_nk, extra = NCHUNK // NWK, NCHUNK % NWK
    nk_max = base_nk + (1 if extra else 0)

    NR = 2 if with_counts else 3
    NSLOT = 2 * NR

    mesh = plsc.VectorSubcoreMesh(core_axis_name="c", subcore_axis_name="s")
    out_type = [jax.ShapeDtypeStruct((NC, N_NODES, CH), jnp.float32)]
    if with_counts:
        out_type.append(jax.ShapeDtypeStruct((NC, N_NODES, 16), jnp.float32))
    scratch = [
        pltpu.VMEM_SHARED((N_NODES, CH), jnp.float32),
        pltpu.VMEM_SHARED((N_NODES, 16), jnp.float32),
        pltpu.VMEM((NSLOT, B), jnp.int32),
        pltpu.VMEM((NSLOT, B), jnp.int32),
        [pltpu.VMEM((B, CH), jnp.float32) for _ in range(NR)],
        pltpu.VMEM((B, 16), jnp.float32),
        [pltpu.SemaphoreType.DMA for _ in range(NR)],
        [pltpu.SemaphoreType.DMA for _ in range(NR)],
        [pltpu.SemaphoreType.DMA for _ in range(2)],
    ]

    def body(x_hbm, src_hbm, dst_hbm, zrow_hbm, zcnt_hbm, ones_hbm,
             *rest):
        if with_counts:
            out_hbm, cnt_hbm = rest[0], rest[1]
            acc, cacc, sidx, didx, rows, obuf, gsem, ssem, isem = rest[2:]
        else:
            out_hbm = rest[0]
            acc, cacc, sidx, didx, rows, obuf, gsem, ssem, isem = rest[1:]
        c = lax.axis_index("c")
        s = lax.axis_index("s")
        w = s * NC + c if not split_channels else s
        start = w * base_nk + jnp.minimum(w, extra)
        nk = base_nk + jnp.where(w < extra, 1, 0) if extra else base_nk
        xsrc = x_hbm.at[c] if split_channels else x_hbm

        pltpu.sync_copy(zrow_hbm, acc.at[pl.ds(s * RPT, RPT)])
        if with_counts:
            pltpu.sync_copy(zcnt_hbm, cacc.at[pl.ds(s * RPT, RPT)])
            pltpu.sync_copy(ones_hbm, obuf)
        pltpu.sync_copy(src_hbm.at[pl.ds(start, NR)], sidx.at[pl.ds(0, NR)])
        pltpu.sync_copy(dst_hbm.at[pl.ds(start, NR)], didx.at[pl.ds(0, NR)])
        for m in range(NR - 1):
            pltpu.async_copy(xsrc.at[sidx.at[m]], rows[m], gsem[m])
        plsc.subcore_barrier()

        def scatter_wait(p, q):
            pltpu.make_async_copy(rows[q], acc.at[didx.at[p]],
                                  ssem[q]).wait()
            if with_counts:
                pltpu.make_async_copy(obuf, cacc.at[didx.at[p]],
                                      ssem[q]).wait()

        def step(j, t):
            q = t % NR

            @pl.when(j < nk)
            def _():
                @pl.when(jnp.logical_and(j >= 1, j + NR - 1 < nk))
                def _():
                    cb = start + j + NR - 1
                    sl = (t + NR - 1) % NSLOT
                    pltpu.make_async_copy(src_hbm.at[cb], sidx.at[sl],
                                          isem[(t - 1) % 2]).wait()
                    pltpu.make_async_copy(dst_hbm.at[cb], didx.at[sl],
                                          isem[(t - 1) % 2]).wait()

                pltpu.make_async_copy(xsrc.at[sidx.at[t]], rows[q],
                                      gsem[q]).wait()
                pltpu.async_copy(rows[q], acc.at[didx.at[t]], ssem[q],
                                 add=True)
                if with_counts:
                    pltpu.async_copy(obuf, cacc.at[didx.at[t]], ssem[q],
                                     add=True)

                @pl.when(j >= 1)
                def _():
                    scatter_wait((t - 1) % NSLOT, (q - 1) % NR)

                @pl.when(j + NR < nk)
                def _():
                    cb = start + j + NR
                    sl = (t + NR) % NSLOT
                    pltpu.async_copy(src_hbm.at[cb], sidx.at[sl],
                                     isem[t % 2])
                    pltpu.async_copy(dst_hbm.at[cb], didx.at[sl],
                                     isem[t % 2])

                @pl.when(j + NR - 1 < nk)
                def _():
                    pltpu.async_copy(
                        xsrc.at[sidx.at[(t + NR - 1) % NSLOT]],
                        rows[(q + NR - 1) % NR], gsem[(q + NR - 1) % NR])

        def unrolled(u, carry):
            for t in range(NSLOT):
                step(u * NSLOT + t, t)
            return carry

        lax.fori_loop(0, (nk_max + NSLOT - 1) // NSLOT, unrolled, 0)

        for q in range(NR):
            @pl.when(((nk - 1) % NR) == q)
            def _(q=q):
                scatter_wait((nk - 1) % NSLOT, q)

        plsc.subcore_barrier()
        pltpu.sync_copy(acc.at[pl.ds(s * RPT, RPT)],
                        out_hbm.at[c].at[pl.ds(s * RPT, RPT)])
        if with_counts:
            pltpu.sync_copy(cacc.at[pl.ds(s * RPT, RPT)],
                            cnt_hbm.at[c].at[pl.ds(s * RPT, RPT)])

    return pl.kernel(body, out_type=out_type, mesh=mesh,
                     scratch_types=scratch,
                     compiler_params=pltpu.CompilerParams(
                         use_tc_tiling_on_sc=False))


_sc_agg_edges_counts = _sc_agg(split_channels=False, with_counts=True)
_sc_agg_edges = _sc_agg(split_channels=False, with_counts=False)
_sc_agg_split = _sc_agg(split_channels=True, with_counts=False)


def _inv_deg(cnt_ref):
    deg = cnt_ref[0] + cnt_ref[1]
    return 1.0 / jnp.clip(deg[:, 0:1], 1.0, None)


BM = 1000
NB = N_NODES // BM


def _layer0_body(agg_ref, cnt_ref, x_ref, wl_ref, bl_ref, wr_ref, out_ref):
    inv = _inv_deg(cnt_ref)
    aggm = (agg_ref[0] + agg_ref[1]) * inv
    res = jnp.dot(aggm, wl_ref[...], preferred_element_type=jnp.float32)
    res = res + jnp.dot(x_ref[...], wr_ref[...],
                        preferred_element_type=jnp.float32)
    res = jnp.maximum(res + bl_ref[...], 0.0)
    out_ref[0] = res[:, :CH]
    out_ref[1] = res[:, CH:]


def _layer0_tc(agg, cnt, x, WlT, bl, WrT):
    return pl.pallas_call(
        _layer0_body,
        grid=(NB,),
        in_specs=[
            pl.BlockSpec((NC, BM, CH), lambda i: (0, i, 0)),
            pl.BlockSpec((NC, BM, 16), lambda i: (0, i, 0)),
            pl.BlockSpec((BM, CH), lambda i: (i, 0)),
            pl.BlockSpec((CH, 2 * CH), lambda i: (0, 0)),
            pl.BlockSpec((1, 2 * CH), lambda i: (0, 0)),
            pl.BlockSpec((CH, 2 * CH), lambda i: (0, 0)),
        ],
        out_specs=pl.BlockSpec((NC, BM, CH), lambda i: (0, i, 0)),
        out_shape=jax.ShapeDtypeStruct((NC, N_NODES, CH), jnp.float32),
    )(agg, cnt, x, WlT, bl, WrT)


def _layer1_body(agg_ref, cnt_ref, x1_ref, wl1_ref, bl1_ref, wr1_ref,
                 wl2_ref, wr2_ref, bl2_ref, h_ref, r2_ref):
    inv = _inv_deg(cnt_ref)
    x2 = jnp.dot(agg_ref[0] * inv, wl1_ref[:CH, :],
                 preferred_element_type=jnp.float32)
    x2 = x2 + jnp.dot(agg_ref[1] * inv, wl1_ref[CH:, :],
                      preferred_element_type=jnp.float32)
    x2 = x2 + jnp.dot(x1_ref[0], wr1_ref[:CH, :],
                      preferred_element_type=jnp.float32)
    x2 = x2 + jnp.dot(x1_ref[1], wr1_ref[CH:, :],
                      preferred_element_type=jnp.float32)
    x2 = jnp.maximum(x2 + bl1_ref[...], 0.0)
    h_ref[...] = jnp.dot(x2, wl2_ref[...], preferred_element_type=jnp.float32)
    r2_ref[...] = jnp.dot(x2, wr2_ref[...],
                          preferred_element_type=jnp.float32) + bl2_ref[...]


def _layer1_tc(agg, cnt, x1s, Wl1T, bl1, Wr1T, Wl2T, Wr2T, bl2):
    return pl.pallas_call(
        _layer1_body,
        grid=(NB,),
        in_specs=[
            pl.BlockSpec((NC, BM, CH), lambda i: (0, i, 0)),
            pl.BlockSpec((NC, BM, 16), lambda i: (0, i, 0)),
            pl.BlockSpec((NC, BM, CH), lambda i: (0, i, 0)),
            pl.BlockSpec((2 * CH, 2 * CH), lambda i: (0, 0)),
            pl.BlockSpec((1, 2 * CH), lambda i: (0, 0)),
            pl.BlockSpec((2 * CH, 2 * CH), lambda i: (0, 0)),
            pl.BlockSpec((2 * CH, CH), lambda i: (0, 0)),
            pl.BlockSpec((2 * CH, CH), lambda i: (0, 0)),
            pl.BlockSpec((1, CH), lambda i: (0, 0)),
        ],
        out_specs=[
            pl.BlockSpec((BM, CH), lambda i: (i, 0)),
            pl.BlockSpec((BM, CH), lambda i: (i, 0)),
        ],
        out_shape=[
            jax.ShapeDtypeStruct((N_NODES, CH), jnp.float32),
            jax.ShapeDtypeStruct((N_NODES, CH), jnp.float32),
        ],
    )(agg, cnt, x1s, Wl1T, bl1, Wr1T, Wl2T, Wr2T, bl2)


def _final_body(p_ref, cnt_ref, r2_ref, out_ref):
    inv = _inv_deg(cnt_ref)
    out_ref[...] = jnp.maximum((p_ref[0] + p_ref[1]) * inv + r2_ref[...], 0.0)


def _final_tc(p, cnt, r2):
    return pl.pallas_call(
        _final_body,
        grid=(NB,),
        in_specs=[
            pl.BlockSpec((NC, BM, CH), lambda i: (0, i, 0)),
            pl.BlockSpec((NC, BM, 16), lambda i: (0, i, 0)),
            pl.BlockSpec((BM, CH), lambda i: (i, 0)),
        ],
        out_specs=pl.BlockSpec((BM, CH), lambda i: (i, 0)),
        out_shape=jax.ShapeDtypeStruct((N_NODES, CH), jnp.float32),
    )(p, cnt, r2)


def kernel(x, edge, num_sampled_nodes, num_sampled_edges,
           Wl0, bl0, Wr0, Wl1, bl1, Wr1, Wl2, bl2, Wr2):
    src = edge[0].reshape(NCHUNK, B)
    dst = edge[1].reshape(NCHUNK, B)
    zrow = jnp.zeros((RPT, CH), jnp.float32)
    zcnt = jnp.zeros((RPT, 16), jnp.float32)
    ones = jnp.ones((B, 16), jnp.float32)

    p0, cnt = _sc_agg_edges_counts(x, src, dst, zrow, zcnt, ones)
    x1s = _layer0_tc(p0, cnt, x, Wl0.T, bl0.reshape(1, -1), Wr0.T)

    (a1,) = _sc_agg_split(x1s, src, dst, zrow, zcnt, ones)
    h, r2 = _layer1_tc(a1, cnt, x1s, Wl1.T, bl1.reshape(1, -1), Wr1.T,
                       Wl2.T, Wr2.T, bl2.reshape(1, -1))

    (p2,) = _sc_agg_edges(h, src, dst, zrow, zcnt, ones)
    return _final_tc(p2, cnt, r2)
